# baseline (speedup 1.0000x reference)
import functools

import jax
import jax.numpy as jnp
from jax import lax
from jax.experimental import pallas as pl
from jax.experimental.pallas import tpu as pltpu
from jax.experimental.pallas import tpu_sc as plsc

ROWS = 1000
COLS = 65536
NW = 32
RPW = 32
RG = 8
NG = RPW // RG
C = 2048
NCH = COLS // C
L = 16


def _sc_body(syn_hbm, img_hbm, out_hbm, feat, buf0, buf1, res, sem0, sem1):
    wid = lax.axis_index("s") * 2 + lax.axis_index("c")
    base = wid * RPW

    img_cp = pltpu.async_copy(img_hbm, feat, sem1)
    s0 = jnp.minimum(base, ROWS - RG)
    pltpu.async_copy(syn_hbm.at[pl.ds(s0, RG), pl.ds(0, C)], buf0, sem0)
    img_cp.wait()

    ones = jnp.full((L,), 1.0, jnp.float32)
    zeros = jnp.zeros((L,), jnp.float32)

    def _thresh(i, _):
        v = feat[pl.ds(i * L, L)]
        feat[pl.ds(i * L, L)] = jnp.where(v > 0.7, ones, zeros)
        return 0

    lax.fori_loop(0, COLS // L, _thresh, 0)

    def _inner(bufc, k, accs):
        def body(j, accs):
            fv = feat[pl.ds(k * C + j * L, L)]
            return tuple(
                accs[r] + bufc[r, pl.ds(j * L, L)] * fv for r in range(RG)
            )

        return lax.fori_loop(0, C // L, body, accs)

    for g in range(NG):
        sg = jnp.minimum(base + g * RG, ROWS - RG)
        sg_next = jnp.minimum(base + (g + 1) * RG, ROWS - RG)
        accs = tuple(jnp.zeros((L,), jnp.float32) for _ in range(RG))

        def _pair(p, accs, sg=sg, sg_next=sg_next):
            k0 = 2 * p
            pltpu.make_async_copy(
                syn_hbm.at[pl.ds(sg, RG), pl.ds(0, C)], buf0, sem0
            ).wait()
            pltpu.async_copy(
                syn_hbm.at[pl.ds(sg, RG), pl.ds((k0 + 1) * C, C)], buf1, sem1
            )
            accs = _inner(buf0, k0, accs)
            pltpu.make_async_copy(
                syn_hbm.at[pl.ds(sg, RG), pl.ds(0, C)], buf1, sem1
            ).wait()

            @pl.when(k0 + 2 < NCH)
            def _():
                pltpu.async_copy(
                    syn_hbm.at[pl.ds(sg, RG), pl.ds((k0 + 2) * C, C)], buf0, sem0
                )

            @pl.when(k0 + 2 >= NCH)
            def _():
                pltpu.async_copy(
                    syn_hbm.at[pl.ds(sg_next, RG), pl.ds(0, C)], buf0, sem0
                )

            return _inner(buf1, k0 + 1, accs)

        accs = lax.fori_loop(0, NCH // 2, _pair, accs)
        for r in range(RG):
            res[g * RG + r] = accs[r]

    pltpu.make_async_copy(
        syn_hbm.at[pl.ds(0, RG), pl.ds(0, C)], buf0, sem0
    ).wait()
    pltpu.sync_copy(res, out_hbm.at[wid])


@functools.partial(jax.jit, static_argnames=())
def _sc_overlaps(basal_synapses, img_flat):
    mesh = plsc.VectorSubcoreMesh(
        core_axis_name="c", subcore_axis_name="s", num_cores=2, num_subcores=16
    )
    return pl.kernel(
        _sc_body,
        out_type=jax.ShapeDtypeStruct((NW, RPW, L), jnp.float32),
        mesh=mesh,
        scratch_types=[
            pltpu.VMEM((COLS,), jnp.float32),
            pltpu.VMEM((RG, C), jnp.float32),
            pltpu.VMEM((RG, C), jnp.float32),
            pltpu.VMEM((RPW, L), jnp.float32),
            pltpu.SemaphoreType.DMA,
            pltpu.SemaphoreType.DMA,
        ],
    )(basal_synapses, img_flat)


def _fold_body(acc_ref, ov_ref, lbl_ref):
    x = acc_ref[...]
    s = jnp.sum(x, axis=1, keepdims=True)
    ov_ref[...] = s[:ROWS, :]
    idx = lax.broadcasted_iota(jnp.int32, (NW * RPW, 1), 0)
    sv = jnp.where(idx < ROWS, s, -1.0)
    m = jnp.max(sv)
    lbl_ref[0] = jnp.min(jnp.where(sv == m, idx, NW * RPW))


def _fold_argmax(acc):
    ov, lbl = pl.pallas_call(
        _fold_body,
        out_specs=[
            pl.BlockSpec(memory_space=pltpu.VMEM),
            pl.BlockSpec(memory_space=pltpu.SMEM),
        ],
        out_shape=[
            jax.ShapeDtypeStruct((ROWS, 1), jnp.float32),
            jax.ShapeDtypeStruct((1,), jnp.int32),
        ],
    )(acc)
    return ov.reshape(ROWS), lbl[0]


def kernel(image, basal_synapses):
    padded = _sc_overlaps(basal_synapses, image.reshape(COLS))
    return _fold_argmax(padded.reshape(NW * RPW, L))


# speedup vs baseline: 1.9416x; 1.9416x over previous
import functools

import jax
import jax.numpy as jnp
from jax import lax
from jax.experimental import pallas as pl
from jax.experimental.pallas import tpu as pltpu
from jax.experimental.pallas import tpu_sc as plsc

ROWS = 1000
COLS = 65536
L = 16

NW = 32
RPW = 8
SC_ROWS = NW * RPW
RG = 8
NG = RPW // RG
C = 2048
NCH = COLS // C
U = 4

TC_START = 240
BLOCK_R = 40
TC_ROWS = ROWS - TC_START


def _sc_body(syn_hbm, img_hbm, out_hbm, feat, buf0, buf1, res, sem0, sem1):
    wid = lax.axis_index("s") * 2 + lax.axis_index("c")
    base = wid * RPW

    img_cp = pltpu.async_copy(img_hbm, feat, sem1)
    s0 = jnp.minimum(base, ROWS - RG)
    pltpu.async_copy(syn_hbm.at[pl.ds(s0, RG), pl.ds(0, C)], buf0, sem0)
    img_cp.wait()

    ones = jnp.full((L,), 1.0, jnp.float32)
    zeros = jnp.zeros((L,), jnp.float32)

    def _thresh(i, _):
        v = feat[pl.ds(i * L, L)]
        feat[pl.ds(i * L, L)] = jnp.where(v > 0.7, ones, zeros)
        return 0

    lax.fori_loop(0, COLS // L, _thresh, 0)

    def _inner(bufc, k, accs):
        def body(j, accs):
            for u in range(U):
                off = (j * U + u) * L
                fv = feat[pl.ds(k * C + off, L)]
                accs = tuple(
                    accs[r] + bufc[r, pl.ds(off, L)] * fv for r in range(RG)
                )
            return accs

        return lax.fori_loop(0, C // (L * U), body, accs)

    for g in range(NG):
        sg = jnp.minimum(base + g * RG, ROWS - RG)
        sg_next = jnp.minimum(base + (g + 1) * RG, ROWS - RG)
        accs = tuple(jnp.zeros((L,), jnp.float32) for _ in range(RG))

        def _pair(p, accs, sg=sg, sg_next=sg_next):
            k0 = 2 * p
            pltpu.make_async_copy(
                syn_hbm.at[pl.ds(sg, RG), pl.ds(0, C)], buf0, sem0
            ).wait()
            pltpu.async_copy(
                syn_hbm.at[pl.ds(sg, RG), pl.ds((k0 + 1) * C, C)], buf1, sem1
            )
            accs = _inner(buf0, k0, accs)
            pltpu.make_async_copy(
                syn_hbm.at[pl.ds(sg, RG), pl.ds(0, C)], buf1, sem1
            ).wait()

            @pl.when(k0 + 2 < NCH)
            def _():
                pltpu.async_copy(
                    syn_hbm.at[pl.ds(sg, RG), pl.ds((k0 + 2) * C, C)], buf0, sem0
                )

            @pl.when(k0 + 2 >= NCH)
            def _():
                pltpu.async_copy(
                    syn_hbm.at[pl.ds(sg_next, RG), pl.ds(0, C)], buf0, sem0
                )

            return _inner(buf1, k0 + 1, accs)

        accs = lax.fori_loop(0, NCH // 2, _pair, accs)
        for r in range(RG):
            res[g * RG + r] = accs[r]

    pltpu.make_async_copy(
        syn_hbm.at[pl.ds(0, RG), pl.ds(0, C)], buf0, sem0
    ).wait()
    pltpu.sync_copy(res, out_hbm.at[wid])


def _sc_overlaps(basal_synapses, img_flat):
    mesh = plsc.VectorSubcoreMesh(
        core_axis_name="c", subcore_axis_name="s", num_cores=2, num_subcores=16
    )
    return pl.kernel(
        _sc_body,
        out_type=jax.ShapeDtypeStruct((NW, RPW, L), jnp.float32),
        mesh=mesh,
        scratch_types=[
            pltpu.VMEM((COLS,), jnp.float32),
            pltpu.VMEM((RG, C), jnp.float32),
            pltpu.VMEM((RG, C), jnp.float32),
            pltpu.VMEM((RPW, L), jnp.float32),
            pltpu.SemaphoreType.DMA,
            pltpu.SemaphoreType.DMA,
        ],
    )(basal_synapses, img_flat)


def _tc_body(img_ref, syn_ref, out_ref):
    feat = (img_ref[...] > 0.7).astype(jnp.float32)
    out_ref[...] = jnp.sum(syn_ref[...] * feat, axis=1, keepdims=True)


def _tc_matvec(image, basal_synapses):
    return pl.pallas_call(
        _tc_body,
        grid=(TC_ROWS // BLOCK_R,),
        in_specs=[
            pl.BlockSpec((1, COLS), lambda i: (0, 0)),
            pl.BlockSpec((BLOCK_R, COLS), lambda i: (i + TC_START // BLOCK_R, 0)),
        ],
        out_specs=pl.BlockSpec((BLOCK_R, 1), lambda i: (i, 0)),
        out_shape=jax.ShapeDtypeStruct((TC_ROWS, 1), jnp.float32),
    )(image, basal_synapses)


def _fold_body(acc_ref, tc_ref, ov_ref, lbl_ref):
    x = acc_ref[...]
    s = jnp.sum(x, axis=1, keepdims=True)
    ov = jnp.concatenate([s[:TC_START, :], tc_ref[...]], axis=0)
    ov_ref[...] = ov
    idx = lax.broadcasted_iota(jnp.int32, (ROWS, 1), 0)
    m = jnp.max(ov)
    lbl_ref[0] = jnp.min(jnp.where(ov == m, idx, ROWS))


def _fold_argmax(acc, tc_part):
    ov, lbl = pl.pallas_call(
        _fold_body,
        out_specs=[
            pl.BlockSpec(memory_space=pltpu.VMEM),
            pl.BlockSpec(memory_space=pltpu.SMEM),
        ],
        out_shape=[
            jax.ShapeDtypeStruct((ROWS, 1), jnp.float32),
            jax.ShapeDtypeStruct((1,), jnp.int32),
        ],
    )(acc, tc_part)
    return ov.reshape(ROWS), lbl[0]


def kernel(image, basal_synapses):
    sc_acc = _sc_overlaps(basal_synapses, image.reshape(COLS))
    tc_part = _tc_matvec(image, basal_synapses)
    return _fold_argmax(sc_acc.reshape(SC_ROWS, L), tc_part)
